# speedup vs baseline: 1.0483x; 1.0483x over previous
import jax
import jax.numpy as jnp
from jax.experimental import pallas as pl

_NEG = -1e30


def _nms_body(hm_ref, nms_ref):
    x = hm_ref[...]
    heat = jnp.clip(jax.nn.sigmoid(x), 1e-4, 1.0 - 1e-4)
    c, h, w = heat.shape
    neg = jnp.full((c, h, 1), _NEG, heat.dtype)
    row = jnp.maximum(heat,
                      jnp.maximum(jnp.concatenate([heat[:, :, 1:], neg], axis=2),
                                  jnp.concatenate([neg, heat[:, :, :-1]], axis=2)))
    negr = jnp.full((c, 1, w), _NEG, heat.dtype)
    hmax = jnp.maximum(row,
                       jnp.maximum(jnp.concatenate([row[:, 1:, :], negr], axis=1),
                                   jnp.concatenate([negr, row[:, :-1, :]], axis=1)))
    nms_ref[...] = jnp.where(hmax == heat, heat, 0.0)


def _nms_pallas(hm3):
    C, H, W = hm3.shape
    BC = 8
    return pl.pallas_call(
        _nms_body,
        grid=(C // BC,),
        in_specs=[pl.BlockSpec((BC, H, W), lambda i: (i, 0, 0))],
        out_specs=pl.BlockSpec((BC, H, W), lambda i: (i, 0, 0)),
        out_shape=jax.ShapeDtypeStruct((C, H, W), jnp.float32),
    )(hm3)


def kernel(hm, wh, reg):
    K = 100
    down_ratio = 4.0
    B, C, H, W = hm.shape
    nms = _nms_pallas(hm.reshape(C, H, W)).reshape(B, C, H, W)
    scores, inds = jax.lax.top_k(nms.reshape(B, -1), K)
    clses = (inds // (H * W)).astype(jnp.int32)
    pix = inds % (H * W)
    ys = (pix // W).astype(jnp.float32)
    xs = (pix % W).astype(jnp.float32)
    reg_f = reg.reshape(B, 2, H * W)
    wh_f = wh.reshape(B, 2, H * W)
    rx = jnp.take_along_axis(reg_f[:, 0, :], pix, axis=1)
    ry = jnp.take_along_axis(reg_f[:, 1, :], pix, axis=1)
    w_ = jnp.take_along_axis(wh_f[:, 0, :], pix, axis=1)
    h_ = jnp.take_along_axis(wh_f[:, 1, :], pix, axis=1)
    xs = xs + rx
    ys = ys + ry
    bboxes = jnp.stack([xs - w_ / 2.0, ys - h_ / 2.0,
                        xs + w_ / 2.0, ys + h_ / 2.0], axis=-1)
    bboxes = bboxes * down_ratio
    det_bboxes = jnp.concatenate([bboxes, scores[..., None]], axis=-1)
    det_bboxes = det_bboxes.reshape(K, 5)
    clses_out = clses.reshape(K)
    return det_bboxes, clses_out


# speedup vs baseline: 10.1361x; 9.6692x over previous
import functools

import jax
import jax.numpy as jnp
from jax import lax
from jax.experimental import pallas as pl
from jax.experimental.pallas import tpu as pltpu
from jax.experimental.pallas import tpu_sc as plsc

_NEG = -1e30
_K = 100

_NB = 2048
_BINSCALE = 2048.0
_PADBASE = 1 << 23
_NW = 32
_PER_W = 64
_NCAND = _NW * _PER_W
_ROWS = 80 * 128
_RPW = _ROWS // _NW
_RPS = _ROWS // 16
_HCAP = 64


def _nms_body(hm_ref, nms_ref, rmax_ref):
    x = hm_ref[...]
    heat = jnp.clip(jax.nn.sigmoid(x), 1e-4, 1.0 - 1e-4)
    c, h, w = heat.shape
    neg = jnp.full((c, h, 1), _NEG, heat.dtype)
    row = jnp.maximum(heat,
                      jnp.maximum(jnp.concatenate([heat[:, :, 1:], neg], axis=2),
                                  jnp.concatenate([neg, heat[:, :, :-1]], axis=2)))
    negr = jnp.full((c, 1, w), _NEG, heat.dtype)
    hmax = jnp.maximum(row,
                       jnp.maximum(jnp.concatenate([row[:, 1:, :], negr], axis=1),
                                   jnp.concatenate([negr, row[:, :-1, :]], axis=1)))
    nms = jnp.where(hmax == heat, heat, 0.0)
    nms_ref[...] = nms
    rmax_ref[...] = jnp.max(nms, axis=2)


def _nms_pallas(hm3):
    C, H, W = hm3.shape
    BC = 8
    return pl.pallas_call(
        _nms_body,
        grid=(C // BC,),
        in_specs=[pl.BlockSpec((BC, H, W), lambda i: (i, 0, 0))],
        out_specs=[pl.BlockSpec((BC, H, W), lambda i: (i, 0, 0)),
                   pl.BlockSpec((BC, H), lambda i: (i, 0))],
        out_shape=[jax.ShapeDtypeStruct((C, H, W), jnp.float32),
                   jax.ShapeDtypeStruct((C, H), jnp.float32)],
    )(hm3)


def _sc_body(rm_hbm, nms_hbm, vals_hbm, idx_hbm,
             rm_v, binb_v, hist_v, slice_v, tmp_v, ghist_v,
             snms_v, rowlist_v, valbuf_v, idxbuf_v, sh_hist, sh_ghist):
    nc = 2
    cid = lax.axis_index("c")
    sid = lax.axis_index("s")
    wid = sid * nc + cid
    lanes = jnp.arange(16, dtype=jnp.int32)

    pltpu.sync_copy(rm_hbm, rm_v.at[pl.ds(0, _ROWS)])

    base = sid * _RPS
    for k in range(_RPS // 16):
        v = rm_v[pl.ds(base + k * 16, 16)]
        b = jnp.clip((v * _BINSCALE).astype(jnp.int32), 0, _NB - 1)
        binb_v[pl.ds(k * 16, 16)] = b
    zero16 = jnp.zeros((16,), jnp.int32)
    for j in range(_NB // 16 + 1):
        hist_v[pl.ds(j * 16, 16)] = zero16

    def _ins(i, carry):
        bv = binb_v[pl.ds(i * 16, 16)]
        for l in range(16):
            ml = lanes == l
            h = plsc.load_gather(hist_v, [bv], mask=ml)
            plsc.store_scatter(hist_v, [bv], h + 1, mask=ml)
        return carry
    lax.fori_loop(0, _RPS // 16, _ins, 0)

    pltpu.sync_copy(hist_v.at[pl.ds(0, _NB)], sh_hist.at[sid])
    plsc.subcore_barrier()
    nbs = _NB // 16
    for j in range(nbs // 16):
        slice_v[pl.ds(j * 16, 16)] = zero16
    for r in range(16):
        pltpu.sync_copy(sh_hist.at[r, pl.ds(sid * nbs, nbs)], tmp_v)
        for j in range(nbs // 16):
            slice_v[pl.ds(j * 16, 16)] = (slice_v[pl.ds(j * 16, 16)]
                                          + tmp_v[pl.ds(j * 16, 16)])
    pltpu.sync_copy(slice_v, sh_ghist.at[pl.ds(sid * nbs, nbs)])
    plsc.subcore_barrier()
    pltpu.sync_copy(sh_ghist, ghist_v)

    def _rc(t, carry):
        acc, bchunk, accab, found = carry
        j = _NB // 16 - 1 - t
        s = jnp.sum(ghist_v[pl.ds(j * 16, 16)])
        acc2 = acc + s
        newly = jnp.logical_and(found == 0, acc2 >= _K)
        bchunk = jnp.where(newly, j, bchunk)
        accab = jnp.where(newly, acc, accab)
        found = jnp.where(newly, 1, found)
        return (acc2, bchunk, accab, found)
    _, bchunk, accab, found = lax.fori_loop(
        0, _NB // 16, _rc, (jnp.int32(0), jnp.int32(0), jnp.int32(0), jnp.int32(0)))

    w = ghist_v[pl.ds(bchunk * 16, 16)]
    sfx = plsc.cumsum(lax.rev(w, (0,)))
    kstar = jnp.max(plsc.all_reduce_ffs(sfx >= (_K - accab)))
    bstar = bchunk * 16 + 15 - kstar
    bstar = jnp.where(found == 0, 0, bstar)
    thresh_s = bstar.astype(jnp.float32) * jnp.float32(1.0 / _BINSCALE)
    thresh_v = jnp.full((16,), thresh_s, dtype=jnp.float32)

    zerof16 = jnp.zeros((16,), jnp.float32)
    for k in range(_HCAP // 16 + 1):
        rowlist_v[pl.ds(k * 16, 16)] = zero16
    for k in range(_PER_W // 16 + 1):
        valbuf_v[pl.ds(k * 16, 16)] = zerof16
        pad = (lanes + (_PADBASE + wid * _PER_W + k * 16)).astype(jnp.float32)
        idxbuf_v[pl.ds(k * 16, 16)] = pad

    nhot = jnp.int32(0)
    for gi in range(_RPW // 32):
        gbase = (wid + gi * _NW) * 32
        pltpu.sync_copy(nms_hbm.at[pl.ds(gbase * 128, 32 * 128)],
                        snms_v.at[pl.ds(gi * 32 * 128, 32 * 128)])
        for k in range(2):
            v = rm_v[pl.ds(gbase + k * 16, 16)]
            m = v >= thresh_v
            mi = m.astype(jnp.int32)
            pos = nhot + plsc.cumsum(mi) - mi
            keep = jnp.logical_and(m, pos < _HCAP)
            lr = lanes + (gi * 32 + k * 16)
            plsc.store_scatter(rowlist_v, [pos], lr, mask=keep)
            nhot = jnp.minimum(nhot + jnp.sum(mi), _HCAP)

    def _ext(t, off):
        lr = rowlist_v[pl.ds(t, 16)][0]
        livev = (lanes * 0 + t) < nhot
        gi2 = lr >> 5
        rglob = (wid + gi2 * _NW) * 32 + (lr - (gi2 << 5))
        for k in range(8):
            v = snms_v[pl.ds(lr * 128 + k * 16, 16)]
            m = jnp.logical_and(v >= thresh_v, livev)
            mi = m.astype(jnp.int32)
            pos = off + plsc.cumsum(mi) - mi
            keep = jnp.logical_and(m, pos < _PER_W)
            plsc.store_scatter(valbuf_v, [pos], v, mask=keep)
            iv = (lanes + (rglob * 128 + k * 16)).astype(jnp.float32)
            plsc.store_scatter(idxbuf_v, [pos], iv, mask=keep)
            off = jnp.minimum(off + jnp.sum(mi), _PER_W)
        return off
    lax.fori_loop(0, _HCAP, _ext, jnp.int32(0))

    pltpu.sync_copy(valbuf_v.at[pl.ds(0, _PER_W)],
                    vals_hbm.at[pl.ds(wid * _PER_W, _PER_W)])
    pltpu.sync_copy(idxbuf_v.at[pl.ds(0, _PER_W)],
                    idx_hbm.at[pl.ds(wid * _PER_W, _PER_W)])


@functools.cache
def _sc_compact_fn():
  return functools.partial(
    pl.kernel,
    out_type=(jax.ShapeDtypeStruct((_NCAND,), jnp.float32),
              jax.ShapeDtypeStruct((_NCAND,), jnp.float32)),
    mesh=plsc.VectorSubcoreMesh(core_axis_name="c", subcore_axis_name="s",
                                num_cores=2, num_subcores=16),
    compiler_params=pltpu.CompilerParams(needs_layout_passes=False),
    scratch_types=[
        pltpu.VMEM((_ROWS + 16,), jnp.float32),
        pltpu.VMEM((_RPS + 16,), jnp.int32),
        pltpu.VMEM((_NB + 16,), jnp.int32),
        pltpu.VMEM((_NB // 16,), jnp.int32),
        pltpu.VMEM((_NB // 16,), jnp.int32),
        pltpu.VMEM((_NB,), jnp.int32),
        pltpu.VMEM((_RPW * 128,), jnp.float32),
        pltpu.VMEM((_HCAP + 16,), jnp.int32),
        pltpu.VMEM((_PER_W + 16,), jnp.float32),
        pltpu.VMEM((_PER_W + 16,), jnp.float32),
        pltpu.VMEM_SHARED((16, _NB), jnp.int32),
        pltpu.VMEM_SHARED((_NB,), jnp.int32),
    ],
  )(_sc_body)


def _sel_body(vals_ref, idx_ref, regf_ref, whf_ref, out_ref):
    nb = _NCAND // 128
    eye = (lax.broadcasted_iota(jnp.int32, (128, 128), 0)
           == lax.broadcasted_iota(jnp.int32, (128, 128), 1)).astype(jnp.float32)

    def col(row):
        return lax.dot_general(eye, row, (((1,), (1,)), ((), ())),
                               preferred_element_type=jnp.float32,
                               precision=lax.Precision.HIGHEST)

    def rowv(c):
        return lax.dot_general(c, eye, (((0,), (0,)), ((), ())),
                               preferred_element_type=jnp.float32,
                               precision=lax.Precision.HIGHEST)

    oiota = lax.broadcasted_iota(jnp.int32, (128, 1), 0).astype(jnp.float32)
    sel = jnp.zeros((128, 2), jnp.float32)
    for a in range(nb):
        va_c = col(vals_ref[a:a + 1, :])
        ia_c = col(idx_ref[a:a + 1, :])
        rank = jnp.zeros((128, 1), jnp.float32)
        for b in range(nb):
            vb_r = vals_ref[b:b + 1, :]
            ib_r = idx_ref[b:b + 1, :]
            gt = (vb_r > va_c).astype(jnp.float32)
            tie = jnp.logical_and(vb_r == va_c, ib_r < ia_c).astype(jnp.float32)
            rank = rank + jnp.sum(gt + tie, axis=1, keepdims=True)
        oh = (oiota == rowv(rank)).astype(jnp.float32)
        fa = jnp.concatenate([va_c, ia_c], axis=1)
        sel = sel + lax.dot_general(oh, fa, (((1,), (0,)), ((), ())),
                                    preferred_element_type=jnp.float32,
                               precision=lax.Precision.HIGHEST)

    val_c = sel[:, 0:1]
    idx_c = sel[:, 1:2]
    clsf = jnp.floor(idx_c * (1.0 / 16384.0))
    pix = idx_c - clsf * 16384.0
    ysf = jnp.floor(pix * (1.0 / 128.0))
    xsf = pix - ysf * 128.0

    piota = lax.broadcasted_iota(jnp.int32, (128, 16384), 1).astype(jnp.float32)
    oh2 = (piota == pix).astype(jnp.float32)
    rg = lax.dot_general(oh2, regf_ref[...], (((1,), (1,)), ((), ())),
                         preferred_element_type=jnp.float32,
                               precision=lax.Precision.HIGHEST)
    ww = lax.dot_general(oh2, whf_ref[...], (((1,), (1,)), ((), ())),
                         preferred_element_type=jnp.float32,
                               precision=lax.Precision.HIGHEST)
    xs = xsf + rg[:, 0:1]
    ys = ysf + rg[:, 1:2]
    w_ = ww[:, 0:1]
    h_ = ww[:, 1:2]
    out_ref[...] = jnp.concatenate(
        [(xs - w_ / 2.0) * 4.0, (ys - h_ / 2.0) * 4.0,
         (xs + w_ / 2.0) * 4.0, (ys + h_ / 2.0) * 4.0,
         val_c, clsf, jnp.zeros((128, 2), jnp.float32)], axis=1)


def _sel_pallas(vals2, idx2, regf, whf):
    return pl.pallas_call(
        _sel_body,
        in_specs=[pl.BlockSpec(vals2.shape, lambda: (0, 0)),
                  pl.BlockSpec(idx2.shape, lambda: (0, 0)),
                  pl.BlockSpec(regf.shape, lambda: (0, 0)),
                  pl.BlockSpec(whf.shape, lambda: (0, 0))],
        out_specs=pl.BlockSpec((128, 8), lambda: (0, 0)),
        out_shape=jax.ShapeDtypeStruct((128, 8), jnp.float32),
    )(vals2, idx2, regf, whf)


def kernel(hm, wh, reg):
    B, C, H, W = hm.shape
    nms, rmax = _nms_pallas(hm.reshape(C, H, W))
    cand_v, cand_i = _sc_compact_fn()(rmax.reshape(-1), nms.reshape(-1))
    out = _sel_pallas(cand_v.reshape(16, 128), cand_i.reshape(16, 128),
                      reg.reshape(2, H * W), wh.reshape(2, H * W))
    det_bboxes = out[:_K, :5]
    clses_out = out[:_K, 5].astype(jnp.int32)
    return det_bboxes, clses_out


# speedup vs baseline: 11.6788x; 1.1522x over previous
import functools

import jax
import jax.numpy as jnp
from jax import lax
from jax.experimental import pallas as pl
from jax.experimental.pallas import tpu as pltpu
from jax.experimental.pallas import tpu_sc as plsc

_NEG = -1e30
_K = 100

_NB = 2048
_BINSCALE = 2048.0
_PADBASE = 1 << 23
_NW = 32
_PER_W = 64
_NCAND = _NW * _PER_W
_ROWS = 80 * 128
_RPW = _ROWS // _NW
_RPS = _ROWS // 16
_HCAP = 32
_NG = 320


def _nms_body(hm_ref, nms_ref, rmax_ref, gmax_ref):
    x = hm_ref[...]
    heat = jnp.clip(jax.nn.sigmoid(x), 1e-4, 1.0 - 1e-4)
    c, h, w = heat.shape
    neg = jnp.full((c, h, 1), _NEG, heat.dtype)
    row = jnp.maximum(heat,
                      jnp.maximum(jnp.concatenate([heat[:, :, 1:], neg], axis=2),
                                  jnp.concatenate([neg, heat[:, :, :-1]], axis=2)))
    negr = jnp.full((c, 1, w), _NEG, heat.dtype)
    hmax = jnp.maximum(row,
                       jnp.maximum(jnp.concatenate([row[:, 1:, :], negr], axis=1),
                                   jnp.concatenate([negr, row[:, :-1, :]], axis=1)))
    nms = jnp.where(hmax == heat, heat, 0.0)
    nms_ref[...] = nms
    rmax = jnp.max(nms, axis=2)
    rmax_ref[...] = rmax
    gmax_ref[...] = jnp.concatenate(
        [jnp.max(rmax[:, g * 32:(g + 1) * 32], axis=1, keepdims=True)
         for g in range(4)], axis=1)


def _nms_pallas(hm3):
    C, H, W = hm3.shape
    BC = 8
    return pl.pallas_call(
        _nms_body,
        grid=(C // BC,),
        in_specs=[pl.BlockSpec((BC, H, W), lambda i: (i, 0, 0))],
        out_specs=[pl.BlockSpec((BC, H, W), lambda i: (i, 0, 0)),
                   pl.BlockSpec((BC, H), lambda i: (i, 0)),
                   pl.BlockSpec((BC, 4), lambda i: (i, 0))],
        out_shape=[jax.ShapeDtypeStruct((C, H, W), jnp.float32),
                   jax.ShapeDtypeStruct((C, H), jnp.float32),
                   jax.ShapeDtypeStruct((C, 4), jnp.float32)],
    )(hm3)


def _sc_body(gm_hbm, rm_hbm, nms_hbm, vals_hbm, idx_hbm,
             gm_v, rm_v, binb_v, hist_v, slice_v, tmp_v, ghist_v,
             rlist_v, rows_v, valbuf_v, idxbuf_v,
             sh_hist, sh_ghist, dsem):
    nc = 2
    cid = lax.axis_index("c")
    sid = lax.axis_index("s")
    wid = sid * nc + cid
    lanes = jnp.arange(16, dtype=jnp.int32)
    zero16 = jnp.zeros((16,), jnp.int32)

    pltpu.sync_copy(gm_hbm, gm_v.at[pl.ds(0, _NG)])
    pltpu.sync_copy(rm_hbm, rm_v.at[pl.ds(0, _ROWS)])

    for k in range(2):
        v = gm_v[pl.ds(sid * 32 + k * 16, 16)]
        b = jnp.clip((v * _BINSCALE).astype(jnp.int32), 0, _NB - 1)
        binb_v[pl.ds(k * 16, 16)] = b
    for j in range(_NB // 16):
        hist_v[pl.ds(j * 16, 16)] = zero16
    ones16 = jnp.ones((16,), jnp.int32)
    for k in range(2):
        bv = binb_v[pl.ds(k * 16, 16)]
        for l in range(16):
            ml = jnp.logical_and(lanes == l, (lanes * 0 + sid) < 10)
            h = plsc.load_gather(hist_v, [bv], mask=ml)
            plsc.store_scatter(hist_v, [bv], h + 1, mask=ml)

    pltpu.sync_copy(hist_v, sh_hist.at[sid])
    plsc.subcore_barrier()
    nbs = _NB // 16
    for j in range(nbs // 16):
        slice_v[pl.ds(j * 16, 16)] = zero16
    for r in range(16):
        pltpu.sync_copy(sh_hist.at[r, pl.ds(sid * nbs, nbs)], tmp_v)
        for j in range(nbs // 16):
            slice_v[pl.ds(j * 16, 16)] = (slice_v[pl.ds(j * 16, 16)]
                                          + tmp_v[pl.ds(j * 16, 16)])
    pltpu.sync_copy(slice_v, sh_ghist.at[pl.ds(sid * nbs, nbs)])
    plsc.subcore_barrier()
    pltpu.sync_copy(sh_ghist, ghist_v)

    def _rc(t, carry):
        acc, bchunk, accab, found = carry
        j = _NB // 16 - 1 - t
        s = jnp.sum(ghist_v[pl.ds(j * 16, 16)])
        acc2 = acc + s
        newly = jnp.logical_and(found == 0, acc2 >= _K)
        bchunk = jnp.where(newly, j, bchunk)
        accab = jnp.where(newly, acc, accab)
        found = jnp.where(newly, 1, found)
        return (acc2, bchunk, accab, found)
    _, bchunk, accab, found = lax.fori_loop(
        0, _NB // 16, _rc, (jnp.int32(0), jnp.int32(0), jnp.int32(0), jnp.int32(0)))

    w = ghist_v[pl.ds(bchunk * 16, 16)]
    sfx = plsc.cumsum(lax.rev(w, (0,)))
    kstar = jnp.max(plsc.all_reduce_ffs(sfx >= (_K - accab)))
    bstar = bchunk * 16 + 15 - kstar
    bstar = jnp.where(found == 0, 0, bstar)
    thresh_s = bstar.astype(jnp.float32) * jnp.float32(1.0 / _BINSCALE)
    thresh_v = jnp.full((16,), thresh_s, dtype=jnp.float32)

    zerof16 = jnp.zeros((16,), jnp.float32)
    for k in range(_HCAP // 16 + 1):
        rlist_v[pl.ds(k * 16, 16)] = zero16 + wid * 16
    for k in range(_PER_W // 16 + 1):
        valbuf_v[pl.ds(k * 16, 16)] = zerof16
        pad = (lanes + (_PADBASE + wid * _PER_W + k * 16)).astype(jnp.float32)
        idxbuf_v[pl.ds(k * 16, 16)] = pad

    nhot = jnp.int32(0)
    for j in range(_ROWS // (16 * _NW)):
        rbase = (wid + _NW * j) * 16
        v = rm_v[pl.ds(rbase, 16)]
        m = v >= thresh_v
        mi = m.astype(jnp.int32)
        pos = nhot + plsc.cumsum(mi) - mi
        keep = jnp.logical_and(m, pos < _HCAP)
        plsc.store_scatter(rlist_v, [pos], lanes + rbase, mask=keep)
        nhot = jnp.minimum(nhot + jnp.sum(mi), _HCAP)

    pltpu.async_copy(nms_hbm.at[rlist_v.at[pl.ds(0, _HCAP)]], rows_v, dsem).wait()

    def _ext(t, off):
        rg = rlist_v[pl.ds(t, 16)][0]
        livev = (lanes * 0 + t) < nhot
        for k in range(8):
            v = rows_v[t, pl.ds(k * 16, 16)]
            m = jnp.logical_and(v >= thresh_v, livev)
            mi = m.astype(jnp.int32)
            pos = off + plsc.cumsum(mi) - mi
            keep = jnp.logical_and(m, pos < _PER_W)
            plsc.store_scatter(valbuf_v, [pos], v, mask=keep)
            iv = (lanes + (rg * 128 + k * 16)).astype(jnp.float32)
            plsc.store_scatter(idxbuf_v, [pos], iv, mask=keep)
            off = jnp.minimum(off + jnp.sum(mi), _PER_W)
        return off
    lax.fori_loop(0, _HCAP, _ext, jnp.int32(0))

    pltpu.sync_copy(valbuf_v.at[pl.ds(0, _PER_W)],
                    vals_hbm.at[pl.ds(wid * _PER_W, _PER_W)])
    pltpu.sync_copy(idxbuf_v.at[pl.ds(0, _PER_W)],
                    idx_hbm.at[pl.ds(wid * _PER_W, _PER_W)])


@functools.cache
def _sc_compact_fn():
  return functools.partial(
    pl.kernel,
    out_type=(jax.ShapeDtypeStruct((_NCAND,), jnp.float32),
              jax.ShapeDtypeStruct((_NCAND,), jnp.float32)),
    mesh=plsc.VectorSubcoreMesh(core_axis_name="c", subcore_axis_name="s",
                                num_cores=2, num_subcores=16),
    compiler_params=pltpu.CompilerParams(needs_layout_passes=False),
    scratch_types=[
        pltpu.VMEM((_NG + 16,), jnp.float32),
        pltpu.VMEM((_ROWS + 16,), jnp.float32),
        pltpu.VMEM((48,), jnp.int32),
        pltpu.VMEM((_NB,), jnp.int32),
        pltpu.VMEM((_NB // 16,), jnp.int32),
        pltpu.VMEM((_NB // 16,), jnp.int32),
        pltpu.VMEM((_NB,), jnp.int32),
        pltpu.VMEM((_HCAP + 16,), jnp.int32),
        pltpu.VMEM((_HCAP, 128), jnp.float32),
        pltpu.VMEM((_PER_W + 16,), jnp.float32),
        pltpu.VMEM((_PER_W + 16,), jnp.float32),
        pltpu.VMEM_SHARED((16, _NB), jnp.int32),
        pltpu.VMEM_SHARED((_NB,), jnp.int32),
        pltpu.SemaphoreType.DMA,
    ],
  )(_sc_body)


def _sel_body(vals_ref, idx_ref, regf_ref, whf_ref, out_ref):
    nb = _NCAND // 128
    eye = (lax.broadcasted_iota(jnp.int32, (128, 128), 0)
           == lax.broadcasted_iota(jnp.int32, (128, 128), 1)).astype(jnp.float32)

    def col(row):
        return lax.dot_general(eye, row, (((1,), (1,)), ((), ())),
                               preferred_element_type=jnp.float32,
                               precision=lax.Precision.HIGHEST)

    def rowv(c):
        return lax.dot_general(c, eye, (((0,), (0,)), ((), ())),
                               preferred_element_type=jnp.float32,
                               precision=lax.Precision.HIGHEST)

    oiota = lax.broadcasted_iota(jnp.int32, (128, 1), 0).astype(jnp.float32)
    sel = jnp.zeros((128, 2), jnp.float32)
    for a in range(nb):
        va_c = col(vals_ref[a:a + 1, :])
        ia_c = col(idx_ref[a:a + 1, :])
        rank = jnp.zeros((128, 1), jnp.float32)
        for b in range(nb):
            vb_r = vals_ref[b:b + 1, :]
            ib_r = idx_ref[b:b + 1, :]
            gt = (vb_r > va_c).astype(jnp.float32)
            tie = jnp.logical_and(vb_r == va_c, ib_r < ia_c).astype(jnp.float32)
            rank = rank + jnp.sum(gt + tie, axis=1, keepdims=True)
        oh = (oiota == rowv(rank)).astype(jnp.float32)
        fa = jnp.concatenate([va_c, ia_c], axis=1)
        sel = sel + lax.dot_general(oh, fa, (((1,), (0,)), ((), ())),
                                    preferred_element_type=jnp.float32,
                               precision=lax.Precision.HIGHEST)

    val_c = sel[:, 0:1]
    idx_c = sel[:, 1:2]
    clsf = jnp.floor(idx_c * (1.0 / 16384.0))
    pix = idx_c - clsf * 16384.0
    ysf = jnp.floor(pix * (1.0 / 128.0))
    xsf = pix - ysf * 128.0

    piota = lax.broadcasted_iota(jnp.int32, (128, 16384), 1).astype(jnp.float32)
    oh2 = (piota == pix).astype(jnp.float32)
    rg = lax.dot_general(oh2, regf_ref[...], (((1,), (1,)), ((), ())),
                         preferred_element_type=jnp.float32,
                               precision=lax.Precision.HIGHEST)
    ww = lax.dot_general(oh2, whf_ref[...], (((1,), (1,)), ((), ())),
                         preferred_element_type=jnp.float32,
                               precision=lax.Precision.HIGHEST)
    xs = xsf + rg[:, 0:1]
    ys = ysf + rg[:, 1:2]
    w_ = ww[:, 0:1]
    h_ = ww[:, 1:2]
    out_ref[...] = jnp.concatenate(
        [(xs - w_ / 2.0) * 4.0, (ys - h_ / 2.0) * 4.0,
         (xs + w_ / 2.0) * 4.0, (ys + h_ / 2.0) * 4.0,
         val_c, clsf, jnp.zeros((128, 2), jnp.float32)], axis=1)


def _sel_pallas(vals2, idx2, regf, whf):
    return pl.pallas_call(
        _sel_body,
        in_specs=[pl.BlockSpec(vals2.shape, lambda: (0, 0)),
                  pl.BlockSpec(idx2.shape, lambda: (0, 0)),
                  pl.BlockSpec(regf.shape, lambda: (0, 0)),
                  pl.BlockSpec(whf.shape, lambda: (0, 0))],
        out_specs=pl.BlockSpec((128, 8), lambda: (0, 0)),
        out_shape=jax.ShapeDtypeStruct((128, 8), jnp.float32),
    )(vals2, idx2, regf, whf)


def kernel(hm, wh, reg):
    B, C, H, W = hm.shape
    nms, rmax, gmax = _nms_pallas(hm.reshape(C, H, W))
    cand_v, cand_i = _sc_compact_fn()(gmax.reshape(-1), rmax.reshape(-1),
                                      nms.reshape(_ROWS, W))
    out = _sel_pallas(cand_v.reshape(16, 128), cand_i.reshape(16, 128),
                      reg.reshape(2, H * W), wh.reshape(2, H * W))
    det_bboxes = out[:_K, :5]
    clses_out = out[:_K, 5].astype(jnp.int32)
    return det_bboxes, clses_out


# speedup vs baseline: 15.5549x; 1.3319x over previous
import functools

import jax
import jax.numpy as jnp
from jax import lax
from jax.experimental import pallas as pl
from jax.experimental.pallas import tpu as pltpu
from jax.experimental.pallas import tpu_sc as plsc

_NEG = -1e30
_K = 100

_NB = 2048
_BINSCALE = 2048.0
_PADBASE = 1 << 23
_NW = 32
_PER_W = 32
_NCAND = _NW * _PER_W
_ROWS = 80 * 128
_RPW = _ROWS // _NW
_RPS = _ROWS // 16
_HCAP = 32
_NG = 320


def _nms_body(hm_ref, nms_ref, rmax_ref, gmax_ref):
    x = hm_ref[...]
    heat = jnp.clip(jax.nn.sigmoid(x), 1e-4, 1.0 - 1e-4)
    c, h, w = heat.shape
    neg = jnp.full((c, h, 1), _NEG, heat.dtype)
    row = jnp.maximum(heat,
                      jnp.maximum(jnp.concatenate([heat[:, :, 1:], neg], axis=2),
                                  jnp.concatenate([neg, heat[:, :, :-1]], axis=2)))
    negr = jnp.full((c, 1, w), _NEG, heat.dtype)
    hmax = jnp.maximum(row,
                       jnp.maximum(jnp.concatenate([row[:, 1:, :], negr], axis=1),
                                   jnp.concatenate([negr, row[:, :-1, :]], axis=1)))
    nms = jnp.where(hmax == heat, heat, 0.0)
    nms_ref[...] = nms
    rmax = jnp.max(nms, axis=2)
    rmax_ref[...] = rmax
    gmax_ref[...] = jnp.concatenate(
        [jnp.max(rmax[:, g * 32:(g + 1) * 32], axis=1, keepdims=True)
         for g in range(4)], axis=1)


def _nms_pallas(hm3):
    C, H, W = hm3.shape
    BC = 8
    return pl.pallas_call(
        _nms_body,
        grid=(C // BC,),
        in_specs=[pl.BlockSpec((BC, H, W), lambda i: (i, 0, 0))],
        out_specs=[pl.BlockSpec((BC, H, W), lambda i: (i, 0, 0)),
                   pl.BlockSpec((BC, H), lambda i: (i, 0)),
                   pl.BlockSpec((BC, 4), lambda i: (i, 0))],
        out_shape=[jax.ShapeDtypeStruct((C, H, W), jnp.float32),
                   jax.ShapeDtypeStruct((C, H), jnp.float32),
                   jax.ShapeDtypeStruct((C, 4), jnp.float32)],
    )(hm3)


def _sc_body(gm_hbm, rm_hbm, nms_hbm, vals_hbm, idx_hbm,
             gm_v, rm_v, binb_v, hist_v, slice_v, tmp_v, ghist_v,
             rlist_v, rows_v, valbuf_v, idxbuf_v,
             sh_hist, sh_ghist, dsem):
    nc = 2
    cid = lax.axis_index("c")
    sid = lax.axis_index("s")
    wid = sid * nc + cid
    lanes = jnp.arange(16, dtype=jnp.int32)
    zero16 = jnp.zeros((16,), jnp.int32)

    pltpu.sync_copy(gm_hbm, gm_v.at[pl.ds(0, _NG)])
    pltpu.sync_copy(rm_hbm, rm_v.at[pl.ds(0, _ROWS)])

    for k in range(2):
        v = gm_v[pl.ds(sid * 32 + k * 16, 16)]
        b = jnp.clip((v * _BINSCALE).astype(jnp.int32), 0, _NB - 1)
        binb_v[pl.ds(k * 16, 16)] = b
    for j in range(_NB // 16):
        hist_v[pl.ds(j * 16, 16)] = zero16
    ones16 = jnp.ones((16,), jnp.int32)
    for k in range(2):
        bv = binb_v[pl.ds(k * 16, 16)]
        for l in range(16):
            ml = jnp.logical_and(lanes == l, (lanes * 0 + sid) < 10)
            h = plsc.load_gather(hist_v, [bv], mask=ml)
            plsc.store_scatter(hist_v, [bv], h + 1, mask=ml)

    pltpu.sync_copy(hist_v, sh_hist.at[sid])
    plsc.subcore_barrier()
    nbs = _NB // 16
    for j in range(nbs // 16):
        slice_v[pl.ds(j * 16, 16)] = zero16
    for r in range(16):
        pltpu.sync_copy(sh_hist.at[r, pl.ds(sid * nbs, nbs)], tmp_v)
        for j in range(nbs // 16):
            slice_v[pl.ds(j * 16, 16)] = (slice_v[pl.ds(j * 16, 16)]
                                          + tmp_v[pl.ds(j * 16, 16)])
    pltpu.sync_copy(slice_v, sh_ghist.at[pl.ds(sid * nbs, nbs)])
    plsc.subcore_barrier()
    pltpu.sync_copy(sh_ghist, ghist_v)

    def _rc(t, carry):
        acc, bchunk, accab, found = carry
        j = _NB // 16 - 1 - t
        s = jnp.sum(ghist_v[pl.ds(j * 16, 16)])
        acc2 = acc + s
        newly = jnp.logical_and(found == 0, acc2 >= _K)
        bchunk = jnp.where(newly, j, bchunk)
        accab = jnp.where(newly, acc, accab)
        found = jnp.where(newly, 1, found)
        return (acc2, bchunk, accab, found)
    _, bchunk, accab, found = lax.fori_loop(
        0, _NB // 16, _rc, (jnp.int32(0), jnp.int32(0), jnp.int32(0), jnp.int32(0)))

    w = ghist_v[pl.ds(bchunk * 16, 16)]
    sfx = plsc.cumsum(lax.rev(w, (0,)))
    kstar = jnp.max(plsc.all_reduce_ffs(sfx >= (_K - accab)))
    bstar = bchunk * 16 + 15 - kstar
    bstar = jnp.where(found == 0, 0, bstar)
    thresh_s = bstar.astype(jnp.float32) * jnp.float32(1.0 / _BINSCALE)
    thresh_v = jnp.full((16,), thresh_s, dtype=jnp.float32)

    zerof16 = jnp.zeros((16,), jnp.float32)
    for k in range(_HCAP // 16 + 1):
        rlist_v[pl.ds(k * 16, 16)] = zero16 + wid * 16
    for k in range(_PER_W // 16 + 1):
        valbuf_v[pl.ds(k * 16, 16)] = zerof16
        pad = (lanes + (_PADBASE + wid * _PER_W + k * 16)).astype(jnp.float32)
        idxbuf_v[pl.ds(k * 16, 16)] = pad

    nhot = jnp.int32(0)
    for j in range(_ROWS // (16 * _NW)):
        rbase = (wid + _NW * j) * 16
        v = rm_v[pl.ds(rbase, 16)]
        m = v >= thresh_v
        mi = m.astype(jnp.int32)
        pos = nhot + plsc.cumsum(mi) - mi
        keep = jnp.logical_and(m, pos < _HCAP)
        plsc.store_scatter(rlist_v, [pos], lanes + rbase, mask=keep)
        nhot = jnp.minimum(nhot + jnp.sum(mi), _HCAP)

    pltpu.async_copy(nms_hbm.at[rlist_v.at[pl.ds(0, _HCAP)]], rows_v, dsem).wait()

    def _ext(t, off):
        rg = rlist_v[pl.ds(t, 16)][0]
        livev = (lanes * 0 + t) < nhot
        for k in range(8):
            v = rows_v[t, pl.ds(k * 16, 16)]
            m = jnp.logical_and(v >= thresh_v, livev)
            mi = m.astype(jnp.int32)
            pos = off + plsc.cumsum(mi) - mi
            keep = jnp.logical_and(m, pos < _PER_W)
            plsc.store_scatter(valbuf_v, [pos], v, mask=keep)
            iv = (lanes + (rg * 128 + k * 16)).astype(jnp.float32)
            plsc.store_scatter(idxbuf_v, [pos], iv, mask=keep)
            off = jnp.minimum(off + jnp.sum(mi), _PER_W)
        return off
    lax.fori_loop(0, _HCAP, _ext, jnp.int32(0))

    pltpu.sync_copy(valbuf_v.at[pl.ds(0, _PER_W)],
                    vals_hbm.at[pl.ds(wid * _PER_W, _PER_W)])
    pltpu.sync_copy(idxbuf_v.at[pl.ds(0, _PER_W)],
                    idx_hbm.at[pl.ds(wid * _PER_W, _PER_W)])


@functools.cache
def _sc_compact_fn():
  return functools.partial(
    pl.kernel,
    out_type=(jax.ShapeDtypeStruct((_NCAND,), jnp.float32),
              jax.ShapeDtypeStruct((_NCAND,), jnp.float32)),
    mesh=plsc.VectorSubcoreMesh(core_axis_name="c", subcore_axis_name="s",
                                num_cores=2, num_subcores=16),
    compiler_params=pltpu.CompilerParams(needs_layout_passes=False),
    scratch_types=[
        pltpu.VMEM((_NG + 16,), jnp.float32),
        pltpu.VMEM((_ROWS + 16,), jnp.float32),
        pltpu.VMEM((48,), jnp.int32),
        pltpu.VMEM((_NB,), jnp.int32),
        pltpu.VMEM((_NB // 16,), jnp.int32),
        pltpu.VMEM((_NB // 16,), jnp.int32),
        pltpu.VMEM((_NB,), jnp.int32),
        pltpu.VMEM((_HCAP + 16,), jnp.int32),
        pltpu.VMEM((_HCAP, 128), jnp.float32),
        pltpu.VMEM((_PER_W + 16,), jnp.float32),
        pltpu.VMEM((_PER_W + 16,), jnp.float32),
        pltpu.VMEM_SHARED((16, _NB), jnp.int32),
        pltpu.VMEM_SHARED((_NB,), jnp.int32),
        pltpu.SemaphoreType.DMA,
    ],
  )(_sc_body)


def _sel_body(vals_ref, idx_ref, rw_ref, out_ref):
    nb = _NCAND // 128
    eye = (lax.broadcasted_iota(jnp.int32, (128, 128), 0)
           == lax.broadcasted_iota(jnp.int32, (128, 128), 1)).astype(jnp.float32)

    def col(row):
        return lax.dot_general(eye, row, (((1,), (1,)), ((), ())),
                               preferred_element_type=jnp.float32,
                               precision=lax.Precision.HIGHEST)

    def rowv(c):
        return lax.dot_general(c, eye, (((0,), (0,)), ((), ())),
                               preferred_element_type=jnp.float32,
                               precision=lax.Precision.HIGHEST)

    oiota = lax.broadcasted_iota(jnp.int32, (128, 1), 0).astype(jnp.float32)
    sel = jnp.zeros((128, 2), jnp.float32)
    for a in range(nb):
        va_c = col(vals_ref[a:a + 1, :])
        ia_c = col(idx_ref[a:a + 1, :])
        rank = jnp.zeros((128, 1), jnp.float32)
        for b in range(nb):
            vb_r = vals_ref[b:b + 1, :]
            ib_r = idx_ref[b:b + 1, :]
            gt = (vb_r > va_c).astype(jnp.float32)
            tie = jnp.logical_and(vb_r == va_c, ib_r < ia_c).astype(jnp.float32)
            rank = rank + jnp.sum(gt + tie, axis=1, keepdims=True)
        oh = (oiota == rowv(rank)).astype(jnp.float32)
        fa = jnp.concatenate([va_c, ia_c], axis=1)
        sel = sel + lax.dot_general(oh, fa, (((1,), (0,)), ((), ())),
                                    preferred_element_type=jnp.float32,
                               precision=lax.Precision.HIGHEST)

    val_c = sel[:, 0:1]
    idx_c = sel[:, 1:2]
    clsf = jnp.floor(idx_c * (1.0 / 16384.0))
    pix = idx_c - clsf * 16384.0
    ysf = jnp.floor(pix * (1.0 / 128.0))
    xsf = pix - ysf * 128.0

    hiota = lax.broadcasted_iota(jnp.int32, (128, 128), 1).astype(jnp.float32)
    ohh = (hiota == ysf).astype(jnp.float32)
    ohw = (hiota == xsf).astype(jnp.float32)
    gat = []
    for c in range(4):
        rows = lax.dot_general(ohh, rw_ref[c], (((1,), (0,)), ((), ())),
                               preferred_element_type=jnp.float32,
                               precision=lax.Precision.HIGHEST)
        gat.append(jnp.sum(rows * ohw, axis=1, keepdims=True))
    xs = xsf + gat[0]
    ys = ysf + gat[1]
    w_ = gat[2]
    h_ = gat[3]
    out_ref[...] = jnp.concatenate(
        [(xs - w_ / 2.0) * 4.0, (ys - h_ / 2.0) * 4.0,
         (xs + w_ / 2.0) * 4.0, (ys + h_ / 2.0) * 4.0,
         val_c, clsf, jnp.zeros((128, 2), jnp.float32)], axis=1)


def _sel_pallas(vals2, idx2, rw4):
    return pl.pallas_call(
        _sel_body,
        in_specs=[pl.BlockSpec(vals2.shape, lambda: (0, 0)),
                  pl.BlockSpec(idx2.shape, lambda: (0, 0)),
                  pl.BlockSpec(rw4.shape, lambda: (0, 0, 0))],
        out_specs=pl.BlockSpec((128, 8), lambda: (0, 0)),
        out_shape=jax.ShapeDtypeStruct((128, 8), jnp.float32),
    )(vals2, idx2, rw4)


def kernel(hm, wh, reg):
    B, C, H, W = hm.shape
    nms, rmax, gmax = _nms_pallas(hm.reshape(C, H, W))
    cand_v, cand_i = _sc_compact_fn()(gmax.reshape(-1), rmax.reshape(-1),
                                      nms.reshape(_ROWS, W))
    rw4 = jnp.concatenate([reg.reshape(2, H, W), wh.reshape(2, H, W)], axis=0)
    out = _sel_pallas(cand_v.reshape(-1, 128), cand_i.reshape(-1, 128), rw4)
    det_bboxes = out[:_K, :5]
    clses_out = out[:_K, 5].astype(jnp.int32)
    return det_bboxes, clses_out


# speedup vs baseline: 15.8630x; 1.0198x over previous
import functools

import jax
import jax.numpy as jnp
from jax import lax
from jax.experimental import pallas as pl
from jax.experimental.pallas import tpu as pltpu
from jax.experimental.pallas import tpu_sc as plsc

_NEG = -1e30
_K = 100

_NB = 512
_BINSCALE = 512.0
_PADBASE = 1 << 23
_NW = 32
_PER_W = 32
_NCAND = _NW * _PER_W
_ROWS = 80 * 128
_RPW = _ROWS // _NW
_RPS = _ROWS // 16
_HCAP = 32
_NG = 320


def _nms_body(hm_ref, nms_ref, rmax_ref, gmax_ref):
    x = hm_ref[...]
    heat = jnp.clip(jax.nn.sigmoid(x), 1e-4, 1.0 - 1e-4)
    c, h, w = heat.shape
    neg = jnp.full((c, h, 1), _NEG, heat.dtype)
    row = jnp.maximum(heat,
                      jnp.maximum(jnp.concatenate([heat[:, :, 1:], neg], axis=2),
                                  jnp.concatenate([neg, heat[:, :, :-1]], axis=2)))
    negr = jnp.full((c, 1, w), _NEG, heat.dtype)
    hmax = jnp.maximum(row,
                       jnp.maximum(jnp.concatenate([row[:, 1:, :], negr], axis=1),
                                   jnp.concatenate([negr, row[:, :-1, :]], axis=1)))
    nms = jnp.where(hmax == heat, heat, 0.0)
    nms_ref[...] = nms
    rmax = jnp.max(nms, axis=2)
    rmax_ref[...] = rmax
    gmax_ref[...] = jnp.concatenate(
        [jnp.max(rmax[:, g * 32:(g + 1) * 32], axis=1, keepdims=True)
         for g in range(4)], axis=1)


def _nms_pallas(hm3):
    C, H, W = hm3.shape
    BC = 8
    return pl.pallas_call(
        _nms_body,
        grid=(C // BC,),
        in_specs=[pl.BlockSpec((BC, H, W), lambda i: (i, 0, 0))],
        out_specs=[pl.BlockSpec((BC, H, W), lambda i: (i, 0, 0)),
                   pl.BlockSpec((BC, H), lambda i: (i, 0)),
                   pl.BlockSpec((BC, 4), lambda i: (i, 0))],
        out_shape=[jax.ShapeDtypeStruct((C, H, W), jnp.float32),
                   jax.ShapeDtypeStruct((C, H), jnp.float32),
                   jax.ShapeDtypeStruct((C, 4), jnp.float32)],
    )(hm3)


def _sc_body(gm_hbm, rm_hbm, nms_hbm, vals_hbm, idx_hbm,
             gm_v, rm_v, binb_v, hist_v, slice_v, tmp_v, ghist_v,
             rlist_v, rows_v, valbuf_v, idxbuf_v,
             sh_hist, sh_ghist, dsem):
    nc = 2
    cid = lax.axis_index("c")
    sid = lax.axis_index("s")
    wid = sid * nc + cid
    lanes = jnp.arange(16, dtype=jnp.int32)
    zero16 = jnp.zeros((16,), jnp.int32)

    pltpu.sync_copy(gm_hbm, gm_v.at[pl.ds(0, _NG)])
    pltpu.sync_copy(rm_hbm, rm_v.at[pl.ds(0, _ROWS)])

    for k in range(2):
        v = gm_v[pl.ds(sid * 32 + k * 16, 16)]
        b = jnp.clip((v * _BINSCALE).astype(jnp.int32), 0, _NB - 1)
        binb_v[pl.ds(k * 16, 16)] = b
    for j in range(_NB // 16):
        hist_v[pl.ds(j * 16, 16)] = zero16
    ones16 = jnp.ones((16,), jnp.int32)
    for k in range(2):
        bv = binb_v[pl.ds(k * 16, 16)]
        for l in range(16):
            ml = jnp.logical_and(lanes == l, (lanes * 0 + sid) < 10)
            h = plsc.load_gather(hist_v, [bv], mask=ml)
            plsc.store_scatter(hist_v, [bv], h + 1, mask=ml)

    pltpu.sync_copy(hist_v, sh_hist.at[sid])
    plsc.subcore_barrier()
    nbs = _NB // 16
    for j in range(nbs // 16):
        slice_v[pl.ds(j * 16, 16)] = zero16
    for r in range(16):
        pltpu.sync_copy(sh_hist.at[r, pl.ds(sid * nbs, nbs)], tmp_v)
        for j in range(nbs // 16):
            slice_v[pl.ds(j * 16, 16)] = (slice_v[pl.ds(j * 16, 16)]
                                          + tmp_v[pl.ds(j * 16, 16)])
    pltpu.sync_copy(slice_v, sh_ghist.at[pl.ds(sid * nbs, nbs)])
    plsc.subcore_barrier()
    pltpu.sync_copy(sh_ghist, ghist_v)

    def _rc(t, carry):
        acc, bchunk, accab, found = carry
        j = _NB // 16 - 1 - t
        s = jnp.sum(ghist_v[pl.ds(j * 16, 16)])
        acc2 = acc + s
        newly = jnp.logical_and(found == 0, acc2 >= _K)
        bchunk = jnp.where(newly, j, bchunk)
        accab = jnp.where(newly, acc, accab)
        found = jnp.where(newly, 1, found)
        return (acc2, bchunk, accab, found)
    _, bchunk, accab, found = lax.fori_loop(
        0, _NB // 16, _rc, (jnp.int32(0), jnp.int32(0), jnp.int32(0), jnp.int32(0)))

    w = ghist_v[pl.ds(bchunk * 16, 16)]
    sfx = plsc.cumsum(lax.rev(w, (0,)))
    kstar = jnp.max(plsc.all_reduce_ffs(sfx >= (_K - accab)))
    bstar = bchunk * 16 + 15 - kstar
    bstar = jnp.where(found == 0, 0, bstar)
    thresh_s = bstar.astype(jnp.float32) * jnp.float32(1.0 / _BINSCALE)
    thresh_v = jnp.full((16,), thresh_s, dtype=jnp.float32)

    zerof16 = jnp.zeros((16,), jnp.float32)
    for k in range(_HCAP // 16 + 1):
        rlist_v[pl.ds(k * 16, 16)] = zero16 + wid * 16
    for k in range(_PER_W // 16 + 1):
        valbuf_v[pl.ds(k * 16, 16)] = zerof16
        pad = (lanes + (_PADBASE + wid * _PER_W + k * 16)).astype(jnp.float32)
        idxbuf_v[pl.ds(k * 16, 16)] = pad

    nhot = jnp.int32(0)
    for j in range(_ROWS // (16 * _NW)):
        rbase = (wid + _NW * j) * 16
        v = rm_v[pl.ds(rbase, 16)]
        m = v >= thresh_v
        mi = m.astype(jnp.int32)
        pos = nhot + plsc.cumsum(mi) - mi
        keep = jnp.logical_and(m, pos < _HCAP)
        plsc.store_scatter(rlist_v, [pos], lanes + rbase, mask=keep)
        nhot = jnp.minimum(nhot + jnp.sum(mi), _HCAP)

    pltpu.async_copy(nms_hbm.at[rlist_v.at[pl.ds(0, _HCAP)]], rows_v, dsem).wait()

    def _ext(t, off):
        rg = rlist_v[pl.ds(t, 16)][0]
        livev = (lanes * 0 + t) < nhot
        for k in range(8):
            v = rows_v[t, pl.ds(k * 16, 16)]
            m = jnp.logical_and(v >= thresh_v, livev)
            mi = m.astype(jnp.int32)
            pos = off + plsc.cumsum(mi) - mi
            keep = jnp.logical_and(m, pos < _PER_W)
            plsc.store_scatter(valbuf_v, [pos], v, mask=keep)
            iv = (lanes + (rg * 128 + k * 16)).astype(jnp.float32)
            plsc.store_scatter(idxbuf_v, [pos], iv, mask=keep)
            off = jnp.minimum(off + jnp.sum(mi), _PER_W)
        return off
    lax.fori_loop(0, _HCAP, _ext, jnp.int32(0))

    pltpu.sync_copy(valbuf_v.at[pl.ds(0, _PER_W)],
                    vals_hbm.at[pl.ds(wid * _PER_W, _PER_W)])
    pltpu.sync_copy(idxbuf_v.at[pl.ds(0, _PER_W)],
                    idx_hbm.at[pl.ds(wid * _PER_W, _PER_W)])


@functools.cache
def _sc_compact_fn():
  return functools.partial(
    pl.kernel,
    out_type=(jax.ShapeDtypeStruct((_NCAND,), jnp.float32),
              jax.ShapeDtypeStruct((_NCAND,), jnp.float32)),
    mesh=plsc.VectorSubcoreMesh(core_axis_name="c", subcore_axis_name="s",
                                num_cores=2, num_subcores=16),
    compiler_params=pltpu.CompilerParams(needs_layout_passes=False),
    scratch_types=[
        pltpu.VMEM((_NG + 16,), jnp.float32),
        pltpu.VMEM((_ROWS + 16,), jnp.float32),
        pltpu.VMEM((48,), jnp.int32),
        pltpu.VMEM((_NB,), jnp.int32),
        pltpu.VMEM((_NB // 16,), jnp.int32),
        pltpu.VMEM((_NB // 16,), jnp.int32),
        pltpu.VMEM((_NB,), jnp.int32),
        pltpu.VMEM((_HCAP + 16,), jnp.int32),
        pltpu.VMEM((_HCAP, 128), jnp.float32),
        pltpu.VMEM((_PER_W + 16,), jnp.float32),
        pltpu.VMEM((_PER_W + 16,), jnp.float32),
        pltpu.VMEM_SHARED((16, _NB), jnp.int32),
        pltpu.VMEM_SHARED((_NB,), jnp.int32),
        pltpu.SemaphoreType.DMA,
    ],
  )(_sc_body)


def _sel_body(vals_ref, idx_ref, rw_ref, out_ref):
    nb = _NCAND // 128
    eye = (lax.broadcasted_iota(jnp.int32, (128, 128), 0)
           == lax.broadcasted_iota(jnp.int32, (128, 128), 1)).astype(jnp.float32)

    def col(row):
        return lax.dot_general(eye, row, (((1,), (1,)), ((), ())),
                               preferred_element_type=jnp.float32,
                               precision=lax.Precision.HIGHEST)

    def rowv(c):
        return lax.dot_general(c, eye, (((0,), (0,)), ((), ())),
                               preferred_element_type=jnp.float32,
                               precision=lax.Precision.HIGHEST)

    oiota = lax.broadcasted_iota(jnp.int32, (128, 1), 0).astype(jnp.float32)
    sel = jnp.zeros((128, 2), jnp.float32)
    for a in range(nb):
        va_c = col(vals_ref[a:a + 1, :])
        ia_c = col(idx_ref[a:a + 1, :])
        rank = jnp.zeros((128, 1), jnp.float32)
        for b in range(nb):
            vb_r = vals_ref[b:b + 1, :]
            ib_r = idx_ref[b:b + 1, :]
            gt = (vb_r > va_c).astype(jnp.float32)
            tie = jnp.logical_and(vb_r == va_c, ib_r < ia_c).astype(jnp.float32)
            rank = rank + jnp.sum(gt + tie, axis=1, keepdims=True)
        oh = (oiota == rowv(rank)).astype(jnp.float32)
        fa = jnp.concatenate([va_c, ia_c], axis=1)
        sel = sel + lax.dot_general(oh, fa, (((1,), (0,)), ((), ())),
                                    preferred_element_type=jnp.float32,
                               precision=lax.Precision.HIGHEST)

    val_c = sel[:, 0:1]
    idx_c = sel[:, 1:2]
    clsf = jnp.floor(idx_c * (1.0 / 16384.0))
    pix = idx_c - clsf * 16384.0
    ysf = jnp.floor(pix * (1.0 / 128.0))
    xsf = pix - ysf * 128.0

    hiota = lax.broadcasted_iota(jnp.int32, (128, 128), 1).astype(jnp.float32)
    ohh = (hiota == ysf).astype(jnp.float32)
    ohw = (hiota == xsf).astype(jnp.float32)
    gat = []
    for c in range(4):
        rows = lax.dot_general(ohh, rw_ref[c], (((1,), (0,)), ((), ())),
                               preferred_element_type=jnp.float32,
                               precision=lax.Precision.HIGHEST)
        gat.append(jnp.sum(rows * ohw, axis=1, keepdims=True))
    xs = xsf + gat[0]
    ys = ysf + gat[1]
    w_ = gat[2]
    h_ = gat[3]
    out_ref[...] = jnp.concatenate(
        [(xs - w_ / 2.0) * 4.0, (ys - h_ / 2.0) * 4.0,
         (xs + w_ / 2.0) * 4.0, (ys + h_ / 2.0) * 4.0,
         val_c, clsf, jnp.zeros((128, 2), jnp.float32)], axis=1)


def _sel_pallas(vals2, idx2, rw4):
    return pl.pallas_call(
        _sel_body,
        in_specs=[pl.BlockSpec(vals2.shape, lambda: (0, 0)),
                  pl.BlockSpec(idx2.shape, lambda: (0, 0)),
                  pl.BlockSpec(rw4.shape, lambda: (0, 0, 0))],
        out_specs=pl.BlockSpec((128, 8), lambda: (0, 0)),
        out_shape=jax.ShapeDtypeStruct((128, 8), jnp.float32),
    )(vals2, idx2, rw4)


def kernel(hm, wh, reg):
    B, C, H, W = hm.shape
    nms, rmax, gmax = _nms_pallas(hm.reshape(C, H, W))
    cand_v, cand_i = _sc_compact_fn()(gmax.reshape(-1), rmax.reshape(-1),
                                      nms.reshape(_ROWS, W))
    rw4 = jnp.concatenate([reg.reshape(2, H, W), wh.reshape(2, H, W)], axis=0)
    out = _sel_pallas(cand_v.reshape(-1, 128), cand_i.reshape(-1, 128), rw4)
    det_bboxes = out[:_K, :5]
    clses_out = out[:_K, 5].astype(jnp.int32)
    return det_bboxes, clses_out
